# baseline (speedup 1.0000x reference)
import functools

import numpy as np
import jax
import jax.numpy as jnp
from jax import lax
from jax.experimental import pallas as pl
from jax.experimental.pallas import tpu as pltpu
from jax.experimental.pallas import tpu_sc as plsc

_LANES = 16
_CHUNK = 128


def _pe_table(seq: int, dim: int) -> jnp.ndarray:
    pos = np.arange(seq, dtype=np.float32)[:, None]
    i = np.arange(0, dim, 2, dtype=np.float32)
    div = np.exp(-np.log(10000.0) * i / dim)
    pe = np.zeros((seq, dim), dtype=np.float32)
    pe[:, 0::2] = np.sin(pos * div)
    pe[:, 1::2] = np.cos(pos * div)
    return jnp.asarray(pe)


@functools.cache
def _build(bs: int, seq: int, dim: int):
    try:
        info = plsc.get_sparse_core_info()
        nc, ns = info.num_cores, info.num_subcores
    except Exception:
        nc, ns = 2, 16
    nw = nc * ns
    per_w = bs // nw
    ch = _CHUNK
    nch = per_w // ch
    assert bs % nw == 0 and per_w % ch == 0 and nch % 2 == 0 and per_w % seq == 0

    mesh = plsc.VectorSubcoreMesh(core_axis_name="c", subcore_axis_name="s")

    @functools.partial(
        pl.kernel,
        out_type=jax.ShapeDtypeStruct((bs, dim), jnp.float32),
        mesh=mesh,
        compiler_params=pltpu.CompilerParams(use_tc_tiling_on_sc=False),
        scratch_types=[
            pltpu.VMEM((per_w,), jnp.int32),
            pltpu.VMEM((seq, dim), jnp.float32),
            pltpu.VMEM((ch, dim), jnp.float32),
            pltpu.VMEM((ch, dim), jnp.float32),
            pltpu.SemaphoreType.DMA,
            pltpu.SemaphoreType.DMA,
            pltpu.SemaphoreType.DMA,
            pltpu.SemaphoreType.DMA,
        ],
    )
    def emb(tok_hbm, pe_hbm, table_hbm, out_hbm,
            idx_v, pe_v, buf0, buf1, gsem0, gsem1, wsem0, wsem1):
        wid = lax.axis_index("s") * nc + lax.axis_index("c")
        base = wid * per_w
        pltpu.sync_copy(tok_hbm.at[pl.ds(base, per_w)], idx_v)
        pltpu.sync_copy(pe_hbm, pe_v)
        bufs = (buf0, buf1)
        gsems = (gsem0, gsem1)
        wsems = (wsem0, wsem1)

        def g_copy(c, b):
            return pltpu.make_async_copy(
                table_hbm.at[idx_v.at[pl.ds(c * ch, ch)]], bufs[b], gsems[b])

        def w_copy(c, b):
            return pltpu.make_async_copy(
                bufs[b], out_hbm.at[pl.ds(base + c * ch, ch)], wsems[b])

        g_copy(0, 0).start()

        @pl.loop(0, nch, step=2)
        def _chunks(g):
            for b in range(2):
                c = g + b
                nb = 1 - b

                @pl.when(c + 1 < nch)
                def _():
                    @pl.when(c >= 1)
                    def _():
                        w_copy(c - 1, nb).wait()
                    g_copy(c + 1, nb).start()

                g_copy(c, b).wait()
                cbase = c * ch

                @pl.loop(0, ch)
                def _rows(r):
                    p = lax.rem(cbase + r, seq)
                    for j in range(dim // _LANES):
                        sl = pl.ds(j * _LANES, _LANES)
                        bufs[b][r, sl] = bufs[b][r, sl] + pe_v[p, sl]

                w_copy(c, b).start()

        w_copy(nch - 2, 0).wait()
        w_copy(nch - 1, 1).wait()

    return emb


def kernel(tokens, table):
    b, s = tokens.shape
    _, d = table.shape
    idx = tokens.reshape(-1).astype(jnp.int32)
    pe = _pe_table(s, d)
    out = _build(b * s, s, d)(idx, pe, table)
    return out.reshape(b, s, d)


# speedup vs baseline: 1.0218x; 1.0218x over previous
import functools

import numpy as np
import jax
import jax.numpy as jnp
from jax import lax
from jax.experimental import pallas as pl
from jax.experimental.pallas import tpu as pltpu
from jax.experimental.pallas import tpu_sc as plsc

_LANES = 16
_CHUNK = 128
_NBUF = 5
_AHEAD = 3


def _pe_table(seq: int, dim: int) -> jnp.ndarray:
    pos = np.arange(seq, dtype=np.float32)[:, None]
    i = np.arange(0, dim, 2, dtype=np.float32)
    div = np.exp(-np.log(10000.0) * i / dim)
    pe = np.zeros((seq, dim), dtype=np.float32)
    pe[:, 0::2] = np.sin(pos * div)
    pe[:, 1::2] = np.cos(pos * div)
    return jnp.asarray(pe)


@functools.cache
def _build(bs: int, seq: int, dim: int):
    try:
        info = plsc.get_sparse_core_info()
        nc, ns = info.num_cores, info.num_subcores
    except Exception:
        nc, ns = 2, 16
    nw = nc * ns
    per_w = bs // nw
    ch = _CHUNK
    nch = per_w // ch
    nbuf = _NBUF
    ahead = _AHEAD
    assert bs % nw == 0 and per_w % ch == 0 and nch % nbuf == 0
    assert per_w % seq == 0 and ahead < nbuf

    mesh = plsc.VectorSubcoreMesh(core_axis_name="c", subcore_axis_name="s")

    @functools.partial(
        pl.kernel,
        out_type=jax.ShapeDtypeStruct((bs, dim), jnp.float32),
        mesh=mesh,
        compiler_params=pltpu.CompilerParams(use_tc_tiling_on_sc=False),
        scratch_types=[
            pltpu.VMEM((per_w,), jnp.int32),
            pltpu.VMEM((seq, dim), jnp.float32),
            [pltpu.VMEM((ch, dim), jnp.float32) for _ in range(nbuf)],
            [pltpu.SemaphoreType.DMA for _ in range(nbuf)],
            [pltpu.SemaphoreType.DMA for _ in range(nbuf)],
        ],
    )
    def emb(tok_hbm, pe_hbm, table_hbm, out_hbm, idx_v, pe_v, bufs, gsems, wsems):
        wid = lax.axis_index("s") * nc + lax.axis_index("c")
        base = wid * per_w
        pltpu.sync_copy(tok_hbm.at[pl.ds(base, per_w)], idx_v)
        pltpu.sync_copy(pe_hbm, pe_v)

        def g_copy(c, b):
            return pltpu.make_async_copy(
                table_hbm.at[idx_v.at[pl.ds(c * ch, ch)]], bufs[b], gsems[b])

        def w_copy(c, b):
            return pltpu.make_async_copy(
                bufs[b], out_hbm.at[pl.ds(base + c * ch, ch)], wsems[b])

        for k in range(ahead):
            g_copy(k, k).start()

        @pl.loop(0, nch, step=nbuf)
        def _chunks(g):
            for b in range(nbuf):
                c = g + b
                sa = (b + ahead) % nbuf

                @pl.when(c + ahead < nch)
                def _():
                    @pl.when(c + ahead >= nbuf)
                    def _():
                        w_copy(c + ahead - nbuf, sa).wait()
                    g_copy(c + ahead, sa).start()

                g_copy(c, b).wait()
                cbase = c * ch

                @pl.loop(0, ch, unroll=8)
                def _rows(r):
                    p = lax.rem(cbase + r, seq)
                    for j in range(dim // _LANES):
                        sl = pl.ds(j * _LANES, _LANES)
                        bufs[b][r, sl] = bufs[b][r, sl] + pe_v[p, sl]

                w_copy(c, b).start()

        for b in range(nbuf):
            w_copy(nch - nbuf + b, b).wait()

    return emb


def kernel(tokens, table):
    b, s = tokens.shape
    _, d = table.shape
    idx = tokens.reshape(-1).astype(jnp.int32)
    pe = _pe_table(s, d)
    out = _build(b * s, s, d)(idx, pe, table)
    return out.reshape(b, s, d)


# speedup vs baseline: 1.0220x; 1.0002x over previous
import functools

import numpy as np
import jax
import jax.numpy as jnp
from jax import lax
from jax.experimental import pallas as pl
from jax.experimental.pallas import tpu as pltpu
from jax.experimental.pallas import tpu_sc as plsc

_LANES = 16
_GCH = 40
_NBUF = 4
_AHEAD = 2


def _pe_table(seq: int, dim: int) -> jnp.ndarray:
    pos = np.arange(seq, dtype=np.float32)[:, None]
    i = np.arange(0, dim, 2, dtype=np.float32)
    div = np.exp(-np.log(10000.0) * i / dim)
    pe = np.zeros((seq, dim), dtype=np.float32)
    pe[:, 0::2] = np.sin(pos * div)
    pe[:, 1::2] = np.cos(pos * div)
    return jnp.asarray(pe)


@functools.cache
def _build(batch: int, seq: int, dim: int):
    try:
        info = plsc.get_sparse_core_info()
        nc, ns = info.num_cores, info.num_subcores
    except Exception:
        nc, ns = 2, 16
    nw = nc * ns
    spw = batch // nw
    nbuf = _NBUF
    ahead = _AHEAD
    gch = _GCH
    ng = seq // gch
    assert batch % nw == 0 and spw % nbuf == 0 and seq % gch == 0
    assert gch % 8 == 0 and gch <= 128 and ahead < nbuf

    mesh = plsc.VectorSubcoreMesh(core_axis_name="c", subcore_axis_name="s")

    @functools.partial(
        pl.kernel,
        out_type=jax.ShapeDtypeStruct((batch, seq, dim), jnp.float32),
        mesh=mesh,
        compiler_params=pltpu.CompilerParams(use_tc_tiling_on_sc=False),
        scratch_types=[
            pltpu.VMEM((spw, seq), jnp.int32),
            pltpu.VMEM((seq, dim), jnp.float32),
            [pltpu.VMEM((seq, dim), jnp.float32) for _ in range(nbuf)],
            [pltpu.SemaphoreType.DMA for _ in range(nbuf)],
            [pltpu.SemaphoreType.DMA for _ in range(nbuf)],
        ],
    )
    def emb(tok_hbm, pe_hbm, table_hbm, out_hbm, idx_v, pe_v, bufs, gsems, wsems):
        wid = lax.axis_index("s") * nc + lax.axis_index("c")
        base = wid * spw
        pltpu.sync_copy(tok_hbm.at[pl.ds(base, spw), :], idx_v)
        pltpu.sync_copy(pe_hbm, pe_v)

        def fire_gathers(s, b):
            for k in range(ng):
                pltpu.async_copy(
                    table_hbm.at[idx_v.at[s, pl.ds(k * gch, gch)]],
                    bufs[b].at[pl.ds(k * gch, gch)],
                    gsems[b])

        def wait_gathers(b):
            pltpu.make_async_copy(pe_hbm, bufs[b], gsems[b]).wait()

        def w_copy(s, b):
            return pltpu.make_async_copy(
                bufs[b], out_hbm.at[base + s], wsems[b])

        for k in range(ahead):
            fire_gathers(k, k)

        @pl.loop(0, spw, step=nbuf)
        def _seqs(g):
            for b in range(nbuf):
                s = g + b
                sa = (b + ahead) % nbuf

                @pl.when(s + ahead < spw)
                def _():
                    @pl.when(s + ahead >= nbuf)
                    def _():
                        w_copy(s + ahead - nbuf, sa).wait()
                    fire_gathers(s + ahead, sa)

                wait_gathers(b)

                @pl.loop(0, seq, unroll=8)
                def _rows(r):
                    for j in range(dim // _LANES):
                        sl = pl.ds(j * _LANES, _LANES)
                        bufs[b][r, sl] = bufs[b][r, sl] + pe_v[r, sl]

                w_copy(s, b).start()

        for b in range(nbuf):
            w_copy(spw - nbuf + b, b).wait()

    return emb


def kernel(tokens, table):
    batch, seq = tokens.shape
    _, dim = table.shape
    pe = _pe_table(seq, dim)
    return _build(batch, seq, dim)(tokens.astype(jnp.int32), pe, table)


# speedup vs baseline: 1.2156x; 1.1895x over previous
import functools

import numpy as np
import jax
import jax.numpy as jnp
from jax import lax
from jax.experimental import pallas as pl
from jax.experimental.pallas import tpu as pltpu
from jax.experimental.pallas import tpu_sc as plsc

_LANES = 16
_GCH = 40
_NBUF = 4
_AHEAD = 2
_PAD = 128


def _pe_table(seq: int, dim: int) -> jnp.ndarray:
    pos = np.arange(seq, dtype=np.float32)[:, None]
    i = np.arange(0, dim, 2, dtype=np.float32)
    div = np.exp(-np.log(10000.0) * i / dim)
    pe = np.zeros((seq, dim), dtype=np.float32)
    pe[:, 0::2] = np.sin(pos * div)
    pe[:, 1::2] = np.cos(pos * div)
    return jnp.asarray(pe.reshape(-1))


@functools.cache
def _build(batch: int, seq: int, dim: int):
    try:
        info = plsc.get_sparse_core_info()
        nc, ns = info.num_cores, info.num_subcores
    except Exception:
        nc, ns = 2, 16
    nw = nc * ns
    spw = batch // nw
    nbuf = _NBUF
    ahead = _AHEAD
    gch = _GCH
    ng = seq // gch
    assert batch % nw == 0 and spw % nbuf == 0 and seq % gch == 0
    assert gch % 8 == 0 and gch <= 128 and ahead < nbuf

    mesh = plsc.VectorSubcoreMesh(core_axis_name="c", subcore_axis_name="s")

    @functools.partial(
        pl.kernel,
        out_type=jax.ShapeDtypeStruct((batch, seq, _PAD), jnp.float32),
        mesh=mesh,
        compiler_params=pltpu.CompilerParams(use_tc_tiling_on_sc=True),
        scratch_types=[
            pltpu.VMEM((spw * seq,), jnp.int32),
            pltpu.VMEM((seq * dim,), jnp.float32),
            [pltpu.VMEM((seq, _PAD), jnp.float32) for _ in range(nbuf)],
            [pltpu.SemaphoreType.DMA for _ in range(nbuf)],
            [pltpu.SemaphoreType.DMA for _ in range(nbuf)],
        ],
    )
    def emb(idx_hbm, pe_hbm, table_hbm, out_hbm, idx_v, pe_v, bufs, gsems, wsems):
        wid = lax.axis_index("s") * nc + lax.axis_index("c")
        base = wid * spw
        pltpu.sync_copy(idx_hbm.at[pl.ds(base * seq, spw * seq)], idx_v)
        pltpu.sync_copy(pe_hbm, pe_v)

        def fire_gathers(s, b):
            for k in range(ng):
                pltpu.async_copy(
                    table_hbm.at[idx_v.at[pl.ds(s * seq + k * gch, gch)]],
                    bufs[b].at[pl.ds(k * gch, gch)],
                    gsems[b])

        def wait_gathers(b):
            pltpu.make_async_copy(
                table_hbm.at[pl.ds(0, seq)], bufs[b], gsems[b]).wait()

        def w_copy(s, b):
            return pltpu.make_async_copy(
                bufs[b], out_hbm.at[base + s], wsems[b])

        for k in range(ahead):
            fire_gathers(k, k)

        @pl.loop(0, spw, step=nbuf)
        def _seqs(g):
            for b in range(nbuf):
                s = g + b
                sa = (b + ahead) % nbuf

                @pl.when(s + ahead < spw)
                def _():
                    @pl.when(s + ahead >= nbuf)
                    def _():
                        w_copy(s + ahead - nbuf, sa).wait()
                    fire_gathers(s + ahead, sa)

                wait_gathers(b)

                @pl.loop(0, seq, unroll=8)
                def _rows(r):
                    for j in range(dim // _LANES):
                        sl = pl.ds(j * _LANES, _LANES)
                        bufs[b][r, sl] = bufs[b][r, sl] + pe_v[pl.ds(r * dim + j * _LANES, _LANES)]

                w_copy(s, b).start()

        for b in range(nbuf):
            w_copy(spw - nbuf + b, b).wait()

    return emb


def kernel(tokens, table):
    batch, seq = tokens.shape
    _, dim = table.shape
    idx = tokens.reshape(-1).astype(jnp.int32)
    pe = _pe_table(seq, dim)
    table_p = jnp.pad(table, ((0, 0), (0, _PAD - dim)))
    out = _build(batch, seq, dim)(idx, pe, table_p)
    return out[:, :, :dim]
